# baseline (speedup 1.0000x reference)
import functools

import jax
import jax.numpy as jnp
from jax import lax
from jax.experimental import pallas as pl
from jax.experimental.pallas import tpu as pltpu
from jax.experimental.pallas import tpu_sc as plsc

_B, _L = 64, 4096
_N = _B * _L
_NC, _NS, _LANES = 2, 16, 16
_NW = _NC * _NS
_CHUNK = _N // _NW
_NSUB = 2
_SUB = _CHUNK // _NSUB

_mesh = plsc.VectorSubcoreMesh(core_axis_name="c", subcore_axis_name="s")


@functools.partial(
    pl.kernel,
    mesh=_mesh,
    out_type=jax.ShapeDtypeStruct((_N,), jnp.uint32),
    scratch_types=[
        pltpu.VMEM((_CHUNK,), jnp.uint32),
        pltpu.VMEM((_CHUNK,), jnp.uint32),
        pltpu.VMEM((_CHUNK,), jnp.uint32),
        pltpu.VMEM((_CHUNK,), jnp.uint32),
        pltpu.SemaphoreType.DMA,
        pltpu.SemaphoreType.DMA,
        pltpu.SemaphoreType.DMA,
    ],
)
def _vote_sc(a_hbm, b_hbm, c_hbm, out_hbm, av, bv, cv, ov,
             sem0, sem1, sem_out):
    wid = lax.axis_index("s") * _NC + lax.axis_index("c")
    base = wid * _CHUNK

    in_sems = (sem0, sem1)
    in_copies = []
    for h in range(_NSUB):
        hbm_sl = pl.ds(base + h * _SUB, _SUB)
        vmem_sl = pl.ds(h * _SUB, _SUB)
        in_copies.append([
            pltpu.async_copy(a_hbm.at[hbm_sl], av.at[vmem_sl], in_sems[h]),
            pltpu.async_copy(b_hbm.at[hbm_sl], bv.at[vmem_sl], in_sems[h]),
            pltpu.async_copy(c_hbm.at[hbm_sl], cv.at[vmem_sl], in_sems[h]),
        ])

    out_copies = []
    for h in range(_NSUB):
        for cp in in_copies[h]:
            cp.wait()

        @plsc.parallel_loop(h * _SUB, (h + 1) * _SUB, _LANES, unroll=2)
        def _body(i):
            sl = pl.ds(i, _LANES)
            a = av[sl]
            b = bv[sl]
            c = cv[sl]
            m = jnp.minimum(jnp.minimum(a, b), c)
            r = jnp.where(b == c, b, m)
            r = jnp.where((a == b) | (a == c), a, r)
            ov[sl] = r

        hbm_sl = pl.ds(base + h * _SUB, _SUB)
        vmem_sl = pl.ds(h * _SUB, _SUB)
        out_copies.append(
            pltpu.async_copy(ov.at[vmem_sl], out_hbm.at[hbm_sl], sem_out))

    for cp in out_copies:
        cp.wait()


def kernel(sot1, sot2, sot3):
    a = lax.bitcast_convert_type(sot1.reshape(_N), jnp.uint32)
    b = lax.bitcast_convert_type(sot2.reshape(_N), jnp.uint32)
    c = lax.bitcast_convert_type(sot3.reshape(_N), jnp.uint32)
    out = _vote_sc(a, b, c)
    return lax.bitcast_convert_type(out, jnp.int32).reshape(_B, _L)


# speedup vs baseline: 1.0179x; 1.0179x over previous
import functools

import jax
import jax.numpy as jnp
from jax import lax
from jax.experimental import pallas as pl
from jax.experimental.pallas import tpu as pltpu
from jax.experimental.pallas import tpu_sc as plsc

_B, _L = 64, 4096
_N = _B * _L
_NC, _NS, _LANES = 2, 16, 16
_NW = _NC * _NS
_CHUNK = _N // _NW
_NHALF = _CHUNK // 2

_mesh = plsc.VectorSubcoreMesh(core_axis_name="c", subcore_axis_name="s")


@functools.partial(
    pl.kernel,
    mesh=_mesh,
    out_type=jax.ShapeDtypeStruct((_N,), jnp.int32),
    scratch_types=[
        pltpu.VMEM((_CHUNK,), jnp.int32),
        pltpu.VMEM((_CHUNK,), jnp.int32),
        pltpu.VMEM((_CHUNK,), jnp.int32),
        pltpu.VMEM((_CHUNK,), jnp.int32),
        pltpu.SemaphoreType.DMA,
        pltpu.SemaphoreType.DMA,
        pltpu.SemaphoreType.DMA,
    ],
)
def _vote_sc(a_hbm, b_hbm, c_hbm, out_hbm, av, bv, cv, ov,
             sem_in0, sem_in1, sem_out):
    wid = lax.axis_index("s") * _NC + lax.axis_index("c")
    base = wid * _CHUNK

    in_sems = (sem_in0, sem_in1)
    in_copies = []
    for h in range(2):
        hbm_sl = pl.ds(base + h * _NHALF, _NHALF)
        vmem_sl = pl.ds(h * _NHALF, _NHALF)
        in_copies.append([
            pltpu.async_copy(a_hbm.at[hbm_sl], av.at[vmem_sl], in_sems[h]),
            pltpu.async_copy(b_hbm.at[hbm_sl], bv.at[vmem_sl], in_sems[h]),
            pltpu.async_copy(c_hbm.at[hbm_sl], cv.at[vmem_sl], in_sems[h]),
        ])

    out_copies = []
    for h in range(2):
        for cp in in_copies[h]:
            cp.wait()

        @plsc.parallel_loop(h * _NHALF, (h + 1) * _NHALF, _LANES, unroll=2)
        def _body(i):
            sl = pl.ds(i, _LANES)
            a = av[sl]
            b = bv[sl]
            c = cv[sl]
            m = jnp.minimum(jnp.minimum(a, b), c)
            r = jnp.where(b == c, b, m)
            r = jnp.where((a == b) | (a == c), a, r)
            ov[sl] = r

        hbm_sl = pl.ds(base + h * _NHALF, _NHALF)
        vmem_sl = pl.ds(h * _NHALF, _NHALF)
        out_copies.append(
            pltpu.async_copy(ov.at[vmem_sl], out_hbm.at[hbm_sl], sem_out))

    for cp in out_copies:
        cp.wait()


def kernel(sot1, sot2, sot3):
    out = _vote_sc(sot1.reshape(_N), sot2.reshape(_N), sot3.reshape(_N))
    return out.reshape(_B, _L)
